# baseline (speedup 1.0000x reference)
import functools

import jax
import jax.numpy as jnp
from jax import lax
from jax.experimental import pallas as pl
from jax.experimental.pallas import tpu as pltpu
from jax.experimental.pallas import tpu_sc as plsc

_L = 16


@functools.lru_cache(maxsize=None)
def _build_gather(N, D, V, A, NC, NS):
    NW = NC * NS
    NPW = N // NW
    HALF = V // 2
    C = 128
    NG = C // _L
    assert NPW % C == 0
    NCH = NPW // C
    mesh = plsc.VectorSubcoreMesh(
        core_axis_name="c", subcore_axis_name="s",
        num_cores=NC, num_subcores=NS)

    @functools.partial(
        pl.kernel,
        out_type=jax.ShapeDtypeStruct((N * D,), jnp.float32),
        mesh=mesh,
        scratch_types=[
            pltpu.VMEM((NPW,), jnp.int32),
            pltpu.VMEM((C,), jnp.int32),
            pltpu.VMEM((C,), jnp.int32),
            pltpu.VMEM((C, 2 * D), jnp.float32),
            pltpu.VMEM((C * D,), jnp.float32),
            pltpu.VMEM((A * D,), jnp.float32),
            pltpu.SemaphoreType.DMA,
        ],
        compiler_params=pltpu.CompilerParams(
            use_tc_tiling_on_sc=True, needs_layout_passes=False),
    )
    def kb(ids_hbm, tab2_hbm, aux_hbm, out_hbm,
           ids_v, tidx_v, hcol_v, buf_v, obuf_v, aux_v, sem):
        wid = lax.axis_index("s") * NC + lax.axis_index("c")
        base = wid * NPW
        pltpu.sync_copy(ids_hbm.at[pl.ds(base, NPW)], ids_v)
        pltpu.sync_copy(aux_hbm, aux_v)

        def chunk(t, carry):
            off = t * C
            acc = jnp.zeros((_L,), jnp.int32)
            for g in range(NG):
                v = ids_v[pl.ds(off + g * _L, _L)]
                sp = (lax.shift_right_logical(v - 1, 31)
                      | lax.shift_right_logical(V - v, 31))
                acc = acc + sp
                mrow = jnp.clip(v - 1, 0, V - 1)
                tidx_v[pl.ds(g * _L, _L)] = lax.shift_right_logical(mrow, 1)
                hcol_v[pl.ds(g * _L, _L)] = (mrow & 1) * D

            pltpu.async_copy(tab2_hbm.at[tidx_v], buf_v, sem).wait()

            for g in range(NG):
                slot = lax.iota(jnp.int32, _L) + g * _L
                sbase = slot * D
                hb = hcol_v[pl.ds(g * _L, _L)]
                for col in range(D):
                    cs = jnp.full((_L,), col, jnp.int32)
                    x = plsc.load_gather(buf_v, [slot, hb + cs])
                    plsc.store_scatter(obuf_v, [sbase + cs], x)

            nsp = acc[0]
            for q in range(1, _L):
                nsp = nsp + acc[q]

            @pl.when(nsp > 0)
            def _fixup():
                def fgrp(g, pos):
                    v = ids_v[pl.ds(off + g * _L, _L)]
                    kb_ = jnp.clip(v - V, 0, A - 1) * D

                    def fcol(c, cs):
                        m = plsc.bitcast(v - 1, jnp.uint32) > jnp.uint32(V - 1)
                        x = plsc.load_gather(aux_v, [kb_ + cs], mask=m)
                        plsc.store_scatter(obuf_v, [pos * D + cs], x, mask=m)
                        return cs + 1

                    lax.fori_loop(0, D, fcol, jnp.zeros((_L,), jnp.int32))
                    return pos + _L

                lax.fori_loop(0, NG, fgrp, lax.iota(jnp.int32, _L))

            pltpu.sync_copy(obuf_v, out_hbm.at[pl.ds((base + off) * D, C * D)])
            return carry

        lax.fori_loop(0, NCH, chunk, 0)

    return kb


def kernel(inputs, main_table, re_lut):
    B, Hh = inputs.shape
    V, D = main_table.shape
    A = re_lut.shape[0] + 2
    N = B * Hh
    ids = inputs.reshape(N).astype(jnp.int32)
    aux = jnp.concatenate(
        [jnp.zeros((2, D), jnp.float32), re_lut.astype(jnp.float32)],
        axis=0).reshape(-1)
    tab2 = main_table.astype(jnp.float32).reshape(V // 2, 2 * D)
    kb = _build_gather(N, D, V, A, 2, 16)
    out = kb(ids, tab2, aux)
    return out.reshape(B, Hh, D)


# speedup vs baseline: 1.2128x; 1.2128x over previous
import functools

import jax
import jax.numpy as jnp
from jax import lax
from jax.experimental import pallas as pl
from jax.experimental.pallas import tpu as pltpu
from jax.experimental.pallas import tpu_sc as plsc

_L = 16


@functools.lru_cache(maxsize=None)
def _build_gather(N, D, V, A, NC, NS):
    NW = NC * NS
    NPW = N // NW
    HALF = V // 2
    C = 128
    NG = C // _L
    assert NPW % C == 0
    NCH = NPW // C
    mesh = plsc.VectorSubcoreMesh(
        core_axis_name="c", subcore_axis_name="s",
        num_cores=NC, num_subcores=NS)

    @functools.partial(
        pl.kernel,
        out_type=jax.ShapeDtypeStruct((N * D,), jnp.float32),
        mesh=mesh,
        scratch_types=[
            pltpu.VMEM((NPW,), jnp.int32),
            pltpu.VMEM((C,), jnp.int32),
            pltpu.VMEM((C,), jnp.int32),
            pltpu.VMEM((C, 2 * D), jnp.float32),
            pltpu.VMEM((C * D,), jnp.float32),
            pltpu.VMEM((A * D,), jnp.float32),
            pltpu.SemaphoreType.DMA,
        ],
        compiler_params=pltpu.CompilerParams(
            use_tc_tiling_on_sc=True, needs_layout_passes=False),
    )
    def kb(ids_hbm, tab2_hbm, aux_hbm, out_hbm,
           ids_v, tidx_v, hcol_v, buf_v, obuf_v, aux_v, sem):
        wid = lax.axis_index("s") * NC + lax.axis_index("c")
        base = wid * NPW
        pltpu.sync_copy(ids_hbm.at[pl.ds(base, NPW)], ids_v)
        pltpu.sync_copy(aux_hbm, aux_v)

        def chunk(t, carry):
            off = t * C
            acc = jnp.zeros((_L,), jnp.int32)
            for g in range(NG):
                v = ids_v[pl.ds(off + g * _L, _L)]
                sp = (lax.shift_right_logical(v - 1, 31)
                      | lax.shift_right_logical(V - v, 31))
                acc = acc + sp
                mrow = jnp.clip(v - 1, 0, V - 1)
                h = lax.shift_right_logical((HALF - 1) - mrow, 31)
                tidx_v[pl.ds(g * _L, _L)] = mrow - h * HALF
                hcol_v[pl.ds(g * _L, _L)] = h * D

            pltpu.async_copy(tab2_hbm.at[tidx_v], buf_v, sem).wait()

            def egrp(g, c2):
                hv = hcol_v[pl.ds(g * _L, _L)]
                for q in range(_L):
                    hb = hv[q]
                    r = g * _L + q
                    for c0 in range(0, D, _L):
                        obuf_v[pl.ds(r * D + c0, _L)] = (
                            buf_v[r, pl.ds(hb + c0, _L)])
                return c2

            lax.fori_loop(0, NG, egrp, 0)

            nsp = acc[0]
            for q in range(1, _L):
                nsp = nsp + acc[q]

            @pl.when(nsp > 0)
            def _fixup():
                def fgrp(g, pos):
                    v = ids_v[pl.ds(off + g * _L, _L)]
                    kb_ = jnp.clip(v - V, 0, A - 1) * D

                    def fcol(c, cs):
                        m = plsc.bitcast(v - 1, jnp.uint32) > jnp.uint32(V - 1)
                        x = plsc.load_gather(aux_v, [kb_ + cs], mask=m)
                        plsc.store_scatter(obuf_v, [pos * D + cs], x, mask=m)
                        return cs + 1

                    lax.fori_loop(0, D, fcol, jnp.zeros((_L,), jnp.int32))
                    return pos + _L

                lax.fori_loop(0, NG, fgrp, lax.iota(jnp.int32, _L))

            pltpu.sync_copy(obuf_v, out_hbm.at[pl.ds((base + off) * D, C * D)])
            return carry

        lax.fori_loop(0, NCH, chunk, 0)

    return kb


def kernel(inputs, main_table, re_lut):
    B, Hh = inputs.shape
    V, D = main_table.shape
    A = re_lut.shape[0] + 2
    N = B * Hh
    ids = inputs.reshape(N).astype(jnp.int32)
    aux = jnp.concatenate(
        [jnp.zeros((2, D), jnp.float32), re_lut.astype(jnp.float32)],
        axis=0).reshape(-1)
    mt = main_table.astype(jnp.float32)
    tab2 = jnp.concatenate([mt[:V // 2], mt[V // 2:]], axis=1)
    kb = _build_gather(N, D, V, A, 2, 16)
    out = kb(ids, tab2, aux)
    return out.reshape(B, Hh, D)


# speedup vs baseline: 1.6109x; 1.3283x over previous
import functools

import jax
import jax.numpy as jnp
from jax import lax
from jax.experimental import pallas as pl
from jax.experimental.pallas import tpu as pltpu
from jax.experimental.pallas import tpu_sc as plsc

_L = 16


@functools.lru_cache(maxsize=None)
def _build(N, D, V, A, NC, NS):
    NW = NC * NS
    NPW = N // NW
    C = 640
    assert NPW % C == 0 and C % 128 == 0 and N % NW == 0
    NCH = NPW // C
    NB = C // 128
    mesh = plsc.VectorSubcoreMesh(
        core_axis_name="c", subcore_axis_name="s",
        num_cores=NC, num_subcores=NS)

    @functools.partial(
        pl.kernel,
        out_type=jax.ShapeDtypeStruct((N, D), jnp.float32),
        mesh=mesh,
        scratch_types=[
            pltpu.VMEM((NPW,), jnp.int32),
            pltpu.VMEM((C,), jnp.int32),
            pltpu.VMEM((C,), jnp.int32),
            pltpu.VMEM((C, D), jnp.float32),
            pltpu.VMEM((C, D), jnp.float32),
            pltpu.VMEM((A, D), jnp.float32),
            pltpu.SemaphoreType.DMA,
            pltpu.SemaphoreType.DMA,
            pltpu.SemaphoreType.DMA,
            pltpu.SemaphoreType.DMA,
        ],
        compiler_params=pltpu.CompilerParams(
            use_tc_tiling_on_sc=False, needs_layout_passes=False),
    )
    def k(ids_hbm, main_hbm, aux_hbm, out_hbm,
          ids_v, idx0_v, idx1_v, buf0_v, buf1_v, aux_v,
          sg0, sg1, sw0, sw1):
        wid = lax.axis_index("s") * NC + lax.axis_index("c")
        base = wid * NPW
        pltpu.sync_copy(ids_hbm.at[pl.ds(base, NPW)], ids_v)
        pltpu.sync_copy(aux_hbm, aux_v)

        idx_v = (idx0_v, idx1_v)
        buf_v = (buf0_v, buf1_v)
        sg = (sg0, sg1)
        sw = (sw0, sw1)
        wb = [None, None]

        for t in range(NCH):
            p = t % 2
            off = t * C
            rb = base + off

            if wb[p] is not None:
                wb[p].wait()
                wb[p] = None

            acc = jnp.zeros((_L,), jnp.int32)
            for g in range(C // _L):
                v = ids_v[pl.ds(off + g * _L, _L)]
                sp = (lax.shift_right_logical(v - 1, 31)
                      | lax.shift_right_logical(V - v, 31))
                acc = acc + sp
                idx_v[p][pl.ds(g * _L, _L)] = jnp.clip(v - 1, 0, V - 1)

            cps = [
                pltpu.async_copy(
                    main_hbm.at[idx_v[p].at[pl.ds(j * 128, 128)]],
                    buf_v[p].at[pl.ds(j * 128, 128)], sg[p])
                for j in range(NB)
            ]
            for cp in cps:
                cp.wait()

            nsp = acc[0]
            for q in range(1, _L):
                nsp = nsp + acc[q]

            @pl.when(nsp > 0)
            def _fixup(off=off, p=p):
                def fgrp(g, pos):
                    v = ids_v[pl.ds(off + g * _L, _L)]
                    kk = jnp.clip(v - V, 0, A - 1)

                    def fcol(c, cs):
                        m = plsc.bitcast(v - 1, jnp.uint32) > jnp.uint32(V - 1)
                        x = plsc.load_gather(aux_v, [kk, cs], mask=m)
                        plsc.store_scatter(buf_v[p], [pos, cs], x, mask=m)
                        return cs + 1

                    lax.fori_loop(0, D, fcol, jnp.zeros((_L,), jnp.int32))
                    return pos + _L

                lax.fori_loop(0, C // _L, fgrp, lax.iota(jnp.int32, _L))

            wb[p] = pltpu.async_copy(
                buf_v[p], out_hbm.at[pl.ds(rb, C)], sw[p])

        for p in range(2):
            if wb[p] is not None:
                wb[p].wait()

    return k


def kernel(inputs, main_table, re_lut):
    B, H = inputs.shape
    V, D = main_table.shape
    A = re_lut.shape[0] + 2
    N = B * H
    ids = inputs.reshape(N).astype(jnp.int32)
    aux = jnp.concatenate(
        [jnp.zeros((2, D), jnp.float32), re_lut.astype(jnp.float32)], axis=0)
    k = _build(N, D, V, A, 2, 16)
    out = k(ids, main_table.astype(jnp.float32), aux)
    return out.reshape(B, H, D)


# speedup vs baseline: 1.6267x; 1.0098x over previous
import functools

import jax
import jax.numpy as jnp
from jax import lax
from jax.experimental import pallas as pl
from jax.experimental.pallas import tpu as pltpu
from jax.experimental.pallas import tpu_sc as plsc

_L = 16


@functools.lru_cache(maxsize=None)
def _build(N, D, V, A, NC, NS):
    NW = NC * NS
    NPW = N // NW
    C = 400
    assert NPW % C == 0 and N % NW == 0
    NCH = NPW // C
    NBB = 8
    BPW = NCH * NBB
    mesh = plsc.VectorSubcoreMesh(
        core_axis_name="c", subcore_axis_name="s",
        num_cores=NC, num_subcores=NS)

    @functools.partial(
        pl.kernel,
        out_type=jax.ShapeDtypeStruct((N // 50, 50, D), jnp.float32),
        mesh=mesh,
        scratch_types=[
            pltpu.VMEM((NPW,), jnp.int32),
            pltpu.VMEM((C,), jnp.int32),
            pltpu.VMEM((C,), jnp.int32),
            pltpu.VMEM((C, D), jnp.float32),
            pltpu.VMEM((C, D), jnp.float32),
            pltpu.VMEM((A, D), jnp.float32),
            pltpu.SemaphoreType.DMA,
            pltpu.SemaphoreType.DMA,
            pltpu.SemaphoreType.DMA,
            pltpu.SemaphoreType.DMA,
        ],
        compiler_params=pltpu.CompilerParams(
            use_tc_tiling_on_sc=False, needs_layout_passes=False),
    )
    def k(ids_hbm, main_hbm, aux_hbm, out_hbm,
          ids_v, idx0_v, idx1_v, buf0_v, buf1_v, aux_v,
          sg0, sg1, sw0, sw1):
        wid = lax.axis_index("s") * NC + lax.axis_index("c")
        base = wid * NPW
        pltpu.sync_copy(ids_hbm.at[pl.ds(base, NPW)], ids_v)
        pltpu.sync_copy(aux_hbm, aux_v)

        idx_v = (idx0_v, idx1_v)
        buf_v = (buf0_v, buf1_v)
        sg = (sg0, sg1)
        sw = (sw0, sw1)
        wb = [None, None]

        for t in range(NCH):
            p = t % 2
            off = t * C
            rb = base + off

            if wb[p] is not None:
                for w_ in wb[p]:
                    w_.wait()
                wb[p] = None

            acc = jnp.zeros((_L,), jnp.int32)
            for g in range(C // _L):
                v = ids_v[pl.ds(off + g * _L, _L)]
                sp = (lax.shift_right_logical(v - 1, 31)
                      | lax.shift_right_logical(V - v, 31))
                acc = acc + sp
                idx_v[p][pl.ds(g * _L, _L)] = jnp.clip(v - 1, 0, V - 1)

            cps = [
                pltpu.async_copy(
                    main_hbm.at[idx_v[p].at[pl.ds(o, n)]],
                    buf_v[p].at[pl.ds(o, n)], sg[p])
                for o, n in ((0, 128), (128, 128), (256, 128), (384, 16))
            ]
            for cp in cps:
                cp.wait()

            nsp = acc[0]
            for q in range(1, _L):
                nsp = nsp + acc[q]

            @pl.when(nsp > 0)
            def _fixup(off=off, p=p):
                def fgrp(g, pos):
                    v = ids_v[pl.ds(off + g * _L, _L)]
                    kk = jnp.clip(v - V, 0, A - 1)

                    def fcol(c, cs):
                        m = plsc.bitcast(v - 1, jnp.uint32) > jnp.uint32(V - 1)
                        x = plsc.load_gather(aux_v, [kk, cs], mask=m)
                        plsc.store_scatter(buf_v[p], [pos, cs], x, mask=m)
                        return cs + 1

                    lax.fori_loop(0, D, fcol, jnp.zeros((_L,), jnp.int32))
                    return pos + _L

                lax.fori_loop(0, C // _L, fgrp, lax.iota(jnp.int32, _L))

            wbs = [
                pltpu.async_copy(
                    buf_v[p].at[pl.ds(j * 50, 50)],
                    out_hbm.at[wid * BPW + t * NBB + j], sw[p])
                for j in range(NBB)
            ]
            wb[p] = wbs

        for p in range(2):
            if wb[p] is not None:
                for w_ in wb[p]:
                    w_.wait()

    return k


def kernel(inputs, main_table, re_lut):
    B, H = inputs.shape
    V, D = main_table.shape
    A = re_lut.shape[0] + 2
    N = B * H
    ids = inputs.reshape(N).astype(jnp.int32)
    aux = jnp.concatenate(
        [jnp.zeros((2, D), jnp.float32), re_lut.astype(jnp.float32)], axis=0)
    k = _build(N, D, V, A, 2, 16)
    out = k(ids, main_table.astype(jnp.float32), aux)
    return out.reshape(B, H, D)


# speedup vs baseline: 1.6394x; 1.0078x over previous
import functools

import jax
import jax.numpy as jnp
from jax import lax
from jax.experimental import pallas as pl
from jax.experimental.pallas import tpu as pltpu
from jax.experimental.pallas import tpu_sc as plsc

_L = 16


@functools.lru_cache(maxsize=None)
def _build(N, D, V, A, NC, NS):
    NW = NC * NS
    NPW = N // NW
    C = 400
    assert NPW % C == 0 and N % NW == 0
    NCH = NPW // C
    NBB = 8
    BPW = NCH * NBB
    mesh = plsc.VectorSubcoreMesh(
        core_axis_name="c", subcore_axis_name="s",
        num_cores=NC, num_subcores=NS)

    @functools.partial(
        pl.kernel,
        out_type=jax.ShapeDtypeStruct((N // 50, 50, D), jnp.float32),
        mesh=mesh,
        scratch_types=[
            pltpu.VMEM((NPW,), jnp.int32),
            pltpu.VMEM((C,), jnp.int32),
            pltpu.VMEM((C,), jnp.int32),
            pltpu.VMEM((C, D), jnp.float32),
            pltpu.VMEM((C, D), jnp.float32),
            pltpu.VMEM((A, D), jnp.float32),
            pltpu.SemaphoreType.DMA,
            pltpu.SemaphoreType.DMA,
            pltpu.SemaphoreType.DMA,
            pltpu.SemaphoreType.DMA,
        ],
        compiler_params=pltpu.CompilerParams(
            use_tc_tiling_on_sc=False, needs_layout_passes=False),
    )
    def k(ids_hbm, main_hbm, aux_hbm, out_hbm,
          ids_v, idx0_v, idx1_v, buf0_v, buf1_v, aux_v,
          sg0, sg1, sw0, sw1):
        wid = lax.axis_index("s") * NC + lax.axis_index("c")
        base = wid * NPW
        pltpu.sync_copy(ids_hbm.at[pl.ds(base, NPW)], ids_v)
        pltpu.sync_copy(aux_hbm, aux_v)

        idx_v = (idx0_v, idx1_v)
        buf_v = (buf0_v, buf1_v)
        sg = (sg0, sg1)
        sw = (sw0, sw1)
        wb = [None, None]
        gth = [None, None]
        accs = [None, None]

        for t in range(NCH + 1):
            p = t % 2
            q = 1 - p

            if t < NCH:
                off = t * C
                if wb[p] is not None:
                    for w_ in wb[p]:
                        w_.wait()
                    wb[p] = None

                acc = jnp.zeros((_L,), jnp.int32)
                for g in range(C // _L):
                    v = ids_v[pl.ds(off + g * _L, _L)]
                    sp = (lax.shift_right_logical(v - 1, 31)
                          | lax.shift_right_logical(V - v, 31))
                    acc = acc + sp
                    idx_v[p][pl.ds(g * _L, _L)] = jnp.clip(v - 1, 0, V - 1)
                accs[p] = acc

                gth[p] = [
                    pltpu.async_copy(
                        main_hbm.at[idx_v[p].at[pl.ds(o, n)]],
                        buf_v[p].at[pl.ds(o, n)], sg[p])
                    for o, n in ((0, 128), (128, 128), (256, 128), (384, 16))
                ]

            if t >= 1:
                toff = (t - 1) * C
                for cp in gth[q]:
                    cp.wait()
                gth[q] = None

                acq = accs[q]
                nsp = acq[0]
                for z in range(1, _L):
                    nsp = nsp + acq[z]

                @pl.when(nsp > 0)
                def _fixup(toff=toff, q=q):
                    def fgrp(g, pos):
                        v = ids_v[pl.ds(toff + g * _L, _L)]
                        kk = jnp.clip(v - V, 0, A - 1)

                        def fcol(c, cs):
                            m = (plsc.bitcast(v - 1, jnp.uint32)
                                 > jnp.uint32(V - 1))
                            x = plsc.load_gather(aux_v, [kk, cs], mask=m)
                            plsc.store_scatter(buf_v[q], [pos, cs], x, mask=m)
                            return cs + 1

                        lax.fori_loop(0, D, fcol, jnp.zeros((_L,), jnp.int32))
                        return pos + _L

                    lax.fori_loop(0, C // _L, fgrp, lax.iota(jnp.int32, _L))

                wb[q] = [
                    pltpu.async_copy(
                        buf_v[q].at[pl.ds(j * 50, 50)],
                        out_hbm.at[wid * BPW + (t - 1) * NBB + j], sw[q])
                    for j in range(NBB)
                ]

        for p in range(2):
            if wb[p] is not None:
                for w_ in wb[p]:
                    w_.wait()

    return k


def kernel(inputs, main_table, re_lut):
    B, H = inputs.shape
    V, D = main_table.shape
    A = re_lut.shape[0] + 2
    N = B * H
    ids = inputs.reshape(N).astype(jnp.int32)
    aux = jnp.concatenate(
        [jnp.zeros((2, D), jnp.float32), re_lut.astype(jnp.float32)], axis=0)
    k = _build(N, D, V, A, 2, 16)
    out = k(ids, main_table.astype(jnp.float32), aux)
    return out.reshape(B, H, D)
